# baseline (speedup 1.0000x reference)
import jax
import jax.numpy as jnp
from jax import lax
from jax.experimental import pallas as pl
from jax.experimental.pallas import tpu as pltpu
from jax.experimental.pallas import tpu_sc as plsc
import functools

VOCAB = 100000
D_CONTEXT = 2048
D_MODEL = 128
B = 64
L = 2048

NC = 2
NS = 16
NW = NC * NS

TOKENS = B * L
PER_W = TOKENS // NW
CHUNK = 32
NCHUNK = PER_W // CHUNK
CPL = L // CHUNK
K = 8

_mesh = plsc.VectorSubcoreMesh(
    core_axis_name="c", subcore_axis_name="s", num_cores=NC, num_subcores=NS
)


@functools.partial(
    pl.kernel,
    out_type=jax.ShapeDtypeStruct((TOKENS, D_MODEL), jnp.float32),
    mesh=_mesh,
    scratch_types=[
        pltpu.VMEM((PER_W,), jnp.int32),
        pltpu.VMEM((K, CHUNK, D_MODEL), jnp.float32),
        pltpu.VMEM_SHARED((D_CONTEXT, D_MODEL), jnp.float32),
        pltpu.SemaphoreType.DMA((K,)),
        pltpu.SemaphoreType.DMA((K,)),
        pltpu.SemaphoreType.DMA((K,)),
    ],
)
def _embed_kernel(x_hbm, tok_hbm, pos_hbm, out_hbm,
                  idx_v, rows_v, pos_sh, isem, gsem, ssem):
    cid = lax.axis_index("c")
    sid = lax.axis_index("s")
    wid = sid * NC + cid
    wbase = wid * PER_W

    @pl.when(sid == 0)
    def _():
        pltpu.sync_copy(pos_hbm, pos_sh)

    pltpu.sync_copy(x_hbm.at[pl.ds(wbase, PER_W)], idx_v)
    plsc.subcore_barrier()

    @pl.loop(0, NCHUNK, step=K)
    def _(g):
        for b in range(K):
            c = g + b
            l0 = lax.rem(c, CPL) * CHUNK

            @pl.when(g > 0)
            def _():
                pltpu.make_async_copy(
                    rows_v.at[b], out_hbm.at[pl.ds(wbase, CHUNK)],
                    ssem.at[b]).wait()

            pltpu.async_copy(pos_sh.at[pl.ds(l0, CHUNK)], rows_v.at[b],
                             isem.at[b])
        for b in range(K):
            c = g + b
            pltpu.make_async_copy(pos_sh.at[pl.ds(0, CHUNK)], rows_v.at[b],
                                  isem.at[b]).wait()
            pltpu.async_copy(tok_hbm.at[idx_v.at[pl.ds(c * CHUNK, CHUNK)]],
                             rows_v.at[b], gsem.at[b], add=True)
        for b in range(K):
            c = g + b
            base = wbase + c * CHUNK
            pltpu.make_async_copy(
                tok_hbm.at[idx_v.at[pl.ds(c * CHUNK, CHUNK)]], rows_v.at[b],
                gsem.at[b]).wait()
            pltpu.async_copy(rows_v.at[b], out_hbm.at[pl.ds(base, CHUNK)],
                             ssem.at[b])

    for b in range(K):
        pltpu.make_async_copy(rows_v.at[b], out_hbm.at[pl.ds(wbase, CHUNK)],
                              ssem.at[b]).wait()


def kernel(x, token_table, pos_table):
    xf = x.reshape(-1).astype(jnp.int32)
    out = _embed_kernel(xf, token_table, pos_table)
    return out.reshape(B, L, D_MODEL)


# speedup vs baseline: 1.0346x; 1.0346x over previous
import jax
import jax.numpy as jnp
from jax import lax
from jax.experimental import pallas as pl
from jax.experimental.pallas import tpu as pltpu
from jax.experimental.pallas import tpu_sc as plsc
import functools

VOCAB = 100000
D_CONTEXT = 2048
D_MODEL = 128
B = 64
L = 2048

NC = 2
NS = 16
NW = NC * NS

TOKENS = B * L
PER_W = TOKENS // NW
CHUNK = 64
NCHUNK = PER_W // CHUNK
CPL = L // CHUNK
K = 8

_mesh = plsc.VectorSubcoreMesh(
    core_axis_name="c", subcore_axis_name="s", num_cores=NC, num_subcores=NS
)


@functools.partial(
    pl.kernel,
    out_type=jax.ShapeDtypeStruct((TOKENS, D_MODEL), jnp.float32),
    mesh=_mesh,
    scratch_types=[
        pltpu.VMEM((PER_W,), jnp.int32),
        pltpu.VMEM((K, CHUNK, D_MODEL), jnp.float32),
        pltpu.VMEM_SHARED((D_CONTEXT, D_MODEL), jnp.float32),
        pltpu.SemaphoreType.DMA((K,)),
        pltpu.SemaphoreType.DMA((K,)),
        pltpu.SemaphoreType.DMA((K,)),
    ],
)
def _embed_kernel(x_hbm, tok_hbm, pos_hbm, out_hbm,
                  idx_v, rows_v, pos_sh, isem, gsem, ssem):
    cid = lax.axis_index("c")
    sid = lax.axis_index("s")
    wid = sid * NC + cid
    wbase = wid * PER_W

    @pl.when(sid == 0)
    def _():
        pltpu.sync_copy(pos_hbm, pos_sh)

    pltpu.sync_copy(x_hbm.at[pl.ds(wbase, PER_W)], idx_v)
    plsc.subcore_barrier()

    @pl.loop(0, NCHUNK, step=K)
    def _(g):
        for b in range(K):
            c = g + b
            l0 = lax.rem(c, CPL) * CHUNK

            @pl.when(g > 0)
            def _():
                pltpu.make_async_copy(
                    rows_v.at[b], out_hbm.at[pl.ds(wbase, CHUNK)],
                    ssem.at[b]).wait()

            pltpu.async_copy(pos_sh.at[pl.ds(l0, CHUNK)], rows_v.at[b],
                             isem.at[b])
        for b in range(K):
            c = g + b
            pltpu.make_async_copy(pos_sh.at[pl.ds(0, CHUNK)], rows_v.at[b],
                                  isem.at[b]).wait()
            pltpu.async_copy(tok_hbm.at[idx_v.at[pl.ds(c * CHUNK, CHUNK)]],
                             rows_v.at[b], gsem.at[b], add=True)
        for b in range(K):
            c = g + b
            base = wbase + c * CHUNK
            pltpu.make_async_copy(
                tok_hbm.at[idx_v.at[pl.ds(c * CHUNK, CHUNK)]], rows_v.at[b],
                gsem.at[b]).wait()
            pltpu.async_copy(rows_v.at[b], out_hbm.at[pl.ds(base, CHUNK)],
                             ssem.at[b])

    for b in range(K):
        pltpu.make_async_copy(rows_v.at[b], out_hbm.at[pl.ds(wbase, CHUNK)],
                              ssem.at[b]).wait()


def kernel(x, token_table, pos_table):
    xf = x.reshape(-1).astype(jnp.int32)
    out = _embed_kernel(xf, token_table, pos_table)
    return out.reshape(B, L, D_MODEL)
